# baseline (speedup 1.0000x reference)
import functools

import jax
import jax.numpy as jnp
from jax import lax
from jax.experimental import pallas as pl
from jax.experimental.pallas import tpu as pltpu, tpu_sc as plsc

BATCH = 16384
VOCAB = 1000000
D = 64
BLK = 20480
NBLK = 49
SLEN = NBLK * BLK
NC = 2
NS = 16
NW = NC * NS
BPW = BATCH // NW
NCHUNK = 4
CHUNK = BPW // NCHUNK


def _mv_body(tu_ref, tv_ref, wu_ref, wv_ref, su_ref, sv_ref,
             su_s, sv_s, sem_u, sem_v):
    i = pl.program_id(0)
    su = jax.lax.dot_general(
        wu_ref[...], tu_ref[...], (((0,), (0,)), ((), ())),
        preferred_element_type=jnp.float32,
    )
    sv = jax.lax.dot_general(
        wv_ref[...], tv_ref[...], (((0,), (0,)), ((), ())),
        preferred_element_type=jnp.float32,
    )
    su_s[...] = su.reshape(BLK)
    sv_s[...] = sv.reshape(BLK)
    cu = pltpu.make_async_copy(su_s, su_ref.at[pl.ds(i * BLK, BLK)], sem_u)
    cv = pltpu.make_async_copy(sv_s, sv_ref.at[pl.ds(i * BLK, BLK)], sem_v)
    cu.start()
    cv.start()
    cu.wait()
    cv.wait()


def _scores(tu, tv, wu, wv):
    return pl.pallas_call(
        _mv_body,
        grid=(NBLK,),
        in_specs=[
            pl.BlockSpec((D, BLK), lambda i: (0, i)),
            pl.BlockSpec((D, BLK), lambda i: (0, i)),
            pl.BlockSpec((D, 1), lambda i: (0, 0)),
            pl.BlockSpec((D, 1), lambda i: (0, 0)),
        ],
        out_specs=[
            pl.BlockSpec(memory_space=pltpu.MemorySpace.HBM),
            pl.BlockSpec(memory_space=pltpu.MemorySpace.HBM),
        ],
        out_shape=[
            jax.ShapeDtypeStruct((SLEN,), jnp.float32),
            jax.ShapeDtypeStruct((SLEN,), jnp.float32),
        ],
        scratch_shapes=[
            pltpu.VMEM((BLK,), jnp.float32),
            pltpu.VMEM((BLK,), jnp.float32),
            pltpu.SemaphoreType.DMA,
            pltpu.SemaphoreType.DMA,
        ],
    )(tu, tv, wu, wv)


_mesh = plsc.VectorSubcoreMesh(
    core_axis_name="c", subcore_axis_name="s", num_cores=NC, num_subcores=NS
)


@functools.partial(
    pl.kernel,
    out_type=jax.ShapeDtypeStruct((BATCH,), jnp.float32),
    mesh=_mesh,
    compiler_params=pltpu.CompilerParams(
        needs_layout_passes=False, use_tc_tiling_on_sc=False
    ),
    scratch_types=[
        pltpu.VMEM((NCHUNK, CHUNK), jnp.int32),
        pltpu.VMEM((NCHUNK, CHUNK), jnp.int32),
        pltpu.VMEM((NCHUNK, CHUNK), jnp.float32),
        pltpu.VMEM((NCHUNK, CHUNK), jnp.float32),
        pltpu.VMEM((16,), jnp.float32),
        pltpu.VMEM((BPW,), jnp.float32),
        pltpu.SemaphoreType.DMA,
    ],
)
def _gather_add(uid_hbm, mid_hbm, su_hbm, sv_hbm, bv_hbm, out_hbm,
                uid_v, mid_v, us_v, vs_v, bv_v, out_v, sem):
    wid = lax.axis_index("s") * NC + lax.axis_index("c")

    pltpu.sync_copy(bv_hbm, bv_v)
    pltpu.sync_copy(uid_hbm.at[wid], uid_v)
    pltpu.sync_copy(mid_hbm.at[wid], mid_v)

    copies = []
    for j in range(NCHUNK):
        copies.append(pltpu.async_copy(su_hbm.at[uid_v.at[j]], us_v.at[j], sem))
        copies.append(pltpu.async_copy(sv_hbm.at[mid_v.at[j]], vs_v.at[j], sem))
    for c in copies:
        c.wait()

    bvec = bv_v[pl.ds(0, 16)]

    def chunk_body(g, carry):
        j = g // (CHUNK // 16)
        kk = g - j * (CHUNK // 16)
        u16 = us_v[j, pl.ds(kk * 16, 16)]
        v16 = vs_v[j, pl.ds(kk * 16, 16)]
        out_v[pl.ds(g * 16, 16)] = u16 + v16 + bvec
        return carry

    lax.fori_loop(0, BPW // 16, chunk_body, 0)
    pltpu.sync_copy(out_v, out_hbm.at[pl.ds(wid * BPW, BPW)])


def kernel(user_id, movie_id, user_table, item_table, W, b):
    uid = user_id.astype(jnp.int32).reshape(NW, NCHUNK, CHUNK)
    mid = movie_id.astype(jnp.int32).reshape(NW, NCHUNK, CHUNK)
    wu = W[:D].reshape(D, 1)
    wv = W[D:].reshape(D, 1)
    bv = jnp.broadcast_to(b, (16,))
    su, sv = _scores(user_table.T, item_table.T, wu, wv)
    out = _gather_add(uid, mid, su, sv, bv)
    return out.reshape(BATCH, 1)
